# baseline (speedup 1.0000x reference)
import functools

import jax
import jax.numpy as jnp
from jax import lax
from jax.experimental import pallas as pl
from jax.experimental.pallas import tpu as pltpu
from jax.experimental.pallas import tpu_sc as plsc


def _proj_body(et_ref, eb_ref, w_ref, b_ref, o_ref):
    top = jnp.dot(et_ref[...], w_ref[...], preferred_element_type=jnp.float32)
    bot = jnp.dot(eb_ref[...], w_ref[...], preferred_element_type=jnp.float32)
    o_ref[...] = jnp.concatenate([top, bot], axis=1) + jnp.concatenate(
        [b_ref[...], b_ref[...]], axis=1
    )


def _project_table_packed(embed_table, W, b):
    V, E = embed_table.shape
    D = W.shape[1]
    H = V // 2
    blk = 2000
    return pl.pallas_call(
        _proj_body,
        grid=(H // blk,),
        in_specs=[
            pl.BlockSpec((blk, E), lambda i: (i, 0)),
            pl.BlockSpec((blk, E), lambda i, _h=H // blk: (i + _h, 0)),
            pl.BlockSpec((E, D), lambda i: (0, 0)),
            pl.BlockSpec((1, D), lambda i: (0, 0)),
        ],
        out_specs=pl.BlockSpec((blk, 2 * D), lambda i: (i, 0)),
        out_shape=jax.ShapeDtypeStruct((H, 2 * D), jnp.float32),
    )(embed_table, embed_table, W, b.reshape(1, D))


@functools.lru_cache(maxsize=None)
def _make_gather_t(V, D, Bt, S):
    info = plsc.get_sparse_core_info()
    NC, NS = info.num_cores, info.num_subcores
    NW = NC * NS
    NB = Bt // NW
    mesh = plsc.VectorSubcoreMesh(core_axis_name="c", subcore_axis_name="s")

    @functools.partial(
        pl.kernel,
        mesh=mesh,
        out_type=jax.ShapeDtypeStruct((S, D, Bt), jnp.float32),
        scratch_types=[
            pltpu.VMEM((S, NB), jnp.int32),
            pltpu.VMEM((NB, D), jnp.float32),
            pltpu.VMEM((NB, D), jnp.float32),
            pltpu.VMEM((D, NB), jnp.float32),
            pltpu.VMEM((D, NB), jnp.float32),
            pltpu.SemaphoreType.DMA,
            pltpu.SemaphoreType.DMA,
        ],
        compiler_params=pltpu.CompilerParams(
            use_tc_tiling_on_sc=False, needs_layout_passes=False
        ),
    )
    def gather_kernel(idx_hbm, table_hbm, out_hbm,
                      idx_v, rows0, rows1, tr0, tr1, sem_g, sem_o):
        wid = lax.axis_index("s") * NC + lax.axis_index("c")
        b0 = wid * NB
        pltpu.sync_copy(idx_hbm.at[:, pl.ds(b0, NB)], idx_v)
        rows = (rows0, rows1)
        trs = (tr0, tr1)
        iotas = [lax.broadcasted_iota(jnp.int32, (16,), 0) + 16 * kb
                 for kb in range(NB // 16)]

        def fire(s, slot):
            pltpu.async_copy(table_hbm.at[idx_v.at[s]], rows[slot], sem_g)

        def drain_g(slot):
            pltpu.make_async_copy(
                table_hbm.at[pl.ds(0, NB)], rows[slot], sem_g
            ).wait()

        def write_out(s, slot):
            pltpu.async_copy(
                trs[slot], out_hbm.at[s, :, pl.ds(b0, NB)], sem_o
            )

        def drain_o(slot):
            pltpu.make_async_copy(
                trs[slot], out_hbm.at[0, :, pl.ds(0, NB)], sem_o
            ).wait()

        def transpose(slot):
            src, dst = rows[slot], trs[slot]

            def dbody(d, dv):
                for kb in range(NB // 16):
                    vals = plsc.load_gather(src, [iotas[kb], dv])
                    plsc.store_scatter(dst, [dv, iotas[kb]], vals)
                return dv + 1

            lax.fori_loop(0, D, dbody, jnp.zeros((16,), jnp.int32))

        fire(0, 0)

        def body(s2, c):
            for par in range(2):
                s = 2 * s2 + par

                @pl.when(s + 1 < S)
                def _():
                    fire(s + 1, 1 - par)

                drain_g(par)

                @pl.when(s >= 2)
                def _():
                    drain_o(par)

                transpose(par)
                write_out(s, par)
            return c

        lax.fori_loop(0, S // 2, body, 0)
        drain_o(0)
        drain_o(1)

    return gather_kernel


def kernel(token_ids, embed_table, W, b):
    Bt, S = token_ids.shape
    V, E = embed_table.shape
    D = W.shape[1]
    H = V // 2

    packed = _project_table_packed(embed_table, W, b)
    table = packed.reshape(V, D)

    ids = token_ids.astype(jnp.int32)
    ids2 = 2 * (ids % H) + ids // H
    idx = ids2.T

    outp = _make_gather_t(V, D, Bt, S)(idx, table)
    return outp.transpose(2, 0, 1)


# speedup vs baseline: 1.6068x; 1.6068x over previous
import functools

import jax
import jax.numpy as jnp
from jax import lax
from jax.experimental import pallas as pl
from jax.experimental.pallas import tpu as pltpu
from jax.experimental.pallas import tpu_sc as plsc


def _proj_body(et_ref, eb_ref, w_ref, b_ref, o_ref):
    top = jnp.dot(et_ref[...], w_ref[...], preferred_element_type=jnp.float32)
    bot = jnp.dot(eb_ref[...], w_ref[...], preferred_element_type=jnp.float32)
    o_ref[...] = jnp.concatenate([top, bot], axis=1) + jnp.concatenate(
        [b_ref[...], b_ref[...]], axis=1
    )


def _project_table_packed(embed_table, W, b):
    V, E = embed_table.shape
    D = W.shape[1]
    H = V // 2
    blk = 2000
    return pl.pallas_call(
        _proj_body,
        grid=(H // blk,),
        in_specs=[
            pl.BlockSpec((blk, E), lambda i: (i, 0)),
            pl.BlockSpec((blk, E), lambda i, _h=H // blk: (i + _h, 0)),
            pl.BlockSpec((E, D), lambda i: (0, 0)),
            pl.BlockSpec((1, D), lambda i: (0, 0)),
        ],
        out_specs=pl.BlockSpec((blk, 2 * D), lambda i: (i, 0)),
        out_shape=jax.ShapeDtypeStruct((H, 2 * D), jnp.float32),
    )(embed_table, embed_table, W, b.reshape(1, D))


@functools.lru_cache(maxsize=None)
def _make_gather_t(V, D, Bt, S):
    info = plsc.get_sparse_core_info()
    NC, NS = info.num_cores, info.num_subcores
    NW = NC * NS
    NB = Bt // NW
    mesh = plsc.VectorSubcoreMesh(core_axis_name="c", subcore_axis_name="s")

    @functools.partial(
        pl.kernel,
        mesh=mesh,
        out_type=jax.ShapeDtypeStruct((S, D, Bt), jnp.float32),
        scratch_types=[
            pltpu.VMEM((S, NB), jnp.int32),
            pltpu.VMEM((NB, D), jnp.float32),
            pltpu.VMEM((NB, D), jnp.float32),
            pltpu.VMEM((D, NB), jnp.float32),
            pltpu.VMEM((D, NB), jnp.float32),
            pltpu.SemaphoreType.DMA,
            pltpu.SemaphoreType.DMA,
        ],
        compiler_params=pltpu.CompilerParams(
            use_tc_tiling_on_sc=False, needs_layout_passes=False
        ),
    )
    def gather_kernel(idx_hbm, table_hbm, out_hbm,
                      idx_v, rows0, rows1, tr0, tr1, sem_g, sem_o):
        wid = lax.axis_index("s") * NC + lax.axis_index("c")
        b0 = wid * NB
        pltpu.sync_copy(idx_hbm.at[:, pl.ds(b0, NB)], idx_v)
        rows = (rows0, rows1)
        trs = (tr0, tr1)
        iotas = [lax.broadcasted_iota(jnp.int32, (16,), 0) + 16 * kb
                 for kb in range(NB // 16)]

        def fire(s, slot):
            pltpu.async_copy(table_hbm.at[idx_v.at[s]], rows[slot], sem_g)

        def drain_g(slot):
            pltpu.make_async_copy(
                table_hbm.at[pl.ds(0, NB)], rows[slot], sem_g
            ).wait()

        def write_out(s, slot):
            pltpu.async_copy(
                trs[slot], out_hbm.at[s, :, pl.ds(b0, NB)], sem_o
            )

        def drain_o(slot):
            pltpu.make_async_copy(
                trs[slot], out_hbm.at[0, :, pl.ds(0, NB)], sem_o
            ).wait()

        def transpose(slot):
            src, dst = rows[slot], trs[slot]

            @plsc.parallel_loop(0, D, carry=jnp.zeros((16,), jnp.int32))
            def dbody(d, dv):
                for kb in range(NB // 16):
                    vals = plsc.load_gather(src, [iotas[kb], dv])
                    plsc.store_scatter(dst, [dv, iotas[kb]], vals)
                return dv + 1

        fire(0, 0)

        def body(s2, c):
            for par in range(2):
                s = 2 * s2 + par

                @pl.when(s + 1 < S)
                def _():
                    fire(s + 1, 1 - par)

                drain_g(par)

                @pl.when(s >= 2)
                def _():
                    drain_o(par)

                transpose(par)
                write_out(s, par)
            return c

        lax.fori_loop(0, S // 2, body, 0)
        drain_o(0)
        drain_o(1)

    return gather_kernel


def kernel(token_ids, embed_table, W, b):
    Bt, S = token_ids.shape
    V, E = embed_table.shape
    D = W.shape[1]
    H = V // 2

    packed = _project_table_packed(embed_table, W, b)
    table = packed.reshape(V, D)

    ids = token_ids.astype(jnp.int32)
    ids2 = 2 * (ids % H) + ids // H
    idx = ids2.T

    outp = _make_gather_t(V, D, Bt, S)(idx, table)
    return outp.transpose(2, 0, 1)
